# baseline (speedup 1.0000x reference)
import jax
import jax.numpy as jnp
from jax.experimental import pallas as pl
from jax.experimental.pallas import tpu as pltpu

B, N, NFEAT, NHID1, NHID2 = 8, 2048, 128, 64, 32
ROW_TILE = 512
CHUNK = 512


def _gcn_kernel(length_ref, x_ref, adj_hbm, W1_ref, b1_ref, W2_ref, b2_ref,
                Wlin_ref, blin_ref, out_ref, adjv, s2_scr, sems, qsems):
    i = pl.program_id(0)

    @pl.when(i == 0)
    def _():
        for q in range(4):
            pltpu.make_async_copy(adj_hbm.at[0, pl.ds(q * CHUNK, CHUNK)],
                                  adjv.at[0, pl.ds(q * CHUNK, CHUNK)],
                                  qsems.at[q]).start()

    @pl.when(i + 1 < B)
    def _():
        pltpu.make_async_copy(adj_hbm.at[i + 1], adjv.at[(i + 1) % 3],
                              sems.at[(i + 1) % 3]).start()

    tile_iota = jax.lax.broadcasted_iota(jnp.int32, (ROW_TILE, 1), 0)

    def _l2_tile(g, L, s2, q):
        r0 = q * ROW_TILE
        z = jnp.dot(adjv[g % 3, pl.ds(r0, ROW_TILE), :], s2,
                    preferred_element_type=jnp.float32) + b2_ref[:]
        z = jnp.maximum(z, 0.0)
        z = jnp.where(tile_iota + r0 < L, z, 0.0)
        return jnp.sum(z, axis=0, keepdims=True)

    def _out_write(g, L, pooled):
        out_ref[pl.ds(g, 1), :] = jnp.dot(
            pooled / L.astype(jnp.float32), Wlin_ref[:],
            preferred_element_type=jnp.float32) + blin_ref[:]

    @pl.when(i == B)
    def _():
        g = B - 1
        L = length_ref[g]
        s2 = s2_scr[g % 2]
        n_tiles = (L + ROW_TILE - 1) // ROW_TILE
        pooled = jax.lax.fori_loop(
            0, n_tiles, lambda t, acc: acc + _l2_tile(g, L, s2, t),
            jnp.zeros((1, NHID2), jnp.float32))
        _out_write(g, L, pooled)

    @pl.when(jnp.logical_and(i >= 1, i < B))
    def _():
        g = i - 1
        L = length_ref[g]
        s2 = s2_scr[g % 2]
        s1 = jnp.dot(x_ref[0], W1_ref[:], preferred_element_type=jnp.float32)
        n_tiles = (L + ROW_TILE - 1) // ROW_TILE

        def mk_branch(k):
            def br():
                pooled = jnp.zeros((1, NHID2), jnp.float32)
                for q in range(k):
                    pooled = pooled + _l2_tile(g, L, s2, q)
                pltpu.make_async_copy(adj_hbm.at[i], adjv.at[i % 3],
                                      sems.at[i % 3]).wait()
                h = jnp.dot(adjv[i % 3], s1,
                            preferred_element_type=jnp.float32) + b1_ref[:]
                h = jnp.maximum(h, 0.0)
                s2_scr[i % 2] = jnp.dot(h, W2_ref[:],
                                        preferred_element_type=jnp.float32)
                _out_write(g, L, pooled)
            return br

        jax.lax.switch(n_tiles - 1,
                       [mk_branch(k) for k in range(1, N // ROW_TILE + 1)])

    @pl.when(i == 0)
    def _():
        s1 = jnp.dot(x_ref[0], W1_ref[:], preferred_element_type=jnp.float32)
        for q in range(4):
            pltpu.make_async_copy(adj_hbm.at[0, pl.ds(q * CHUNK, CHUNK)],
                                  adjv.at[0, pl.ds(q * CHUNK, CHUNK)],
                                  qsems.at[q]).wait()
            hq = jnp.dot(adjv[0, q * CHUNK:(q + 1) * CHUNK, :], s1,
                         preferred_element_type=jnp.float32) + b1_ref[:]
            hq = jnp.maximum(hq, 0.0)
            s2_scr[0, q * CHUNK:(q + 1) * CHUNK, :] = jnp.dot(
                hq, W2_ref[:], preferred_element_type=jnp.float32)


def kernel(x, adj, length, W1, b1, W2, b2, Wlin, blin):
    b1r = b1.reshape(1, NHID1)
    b2r = b2.reshape(1, NHID2)
    blinr = blin.reshape(1, 1)

    grid_spec = pltpu.PrefetchScalarGridSpec(
        num_scalar_prefetch=1,
        grid=(B + 1,),
        in_specs=[
            pl.BlockSpec((1, N, NFEAT), lambda i, L: (jnp.minimum(i, B - 1), 0, 0)),
            pl.BlockSpec(memory_space=pl.ANY),
            pl.BlockSpec((NFEAT, NHID1), lambda i, L: (0, 0)),
            pl.BlockSpec((1, NHID1), lambda i, L: (0, 0)),
            pl.BlockSpec((NHID1, NHID2), lambda i, L: (0, 0)),
            pl.BlockSpec((1, NHID2), lambda i, L: (0, 0)),
            pl.BlockSpec((NHID2, 1), lambda i, L: (0, 0)),
            pl.BlockSpec((1, 1), lambda i, L: (0, 0)),
        ],
        out_specs=pl.BlockSpec((B, 1), lambda i, L: (0, 0)),
        scratch_shapes=[
            pltpu.VMEM((3, N, N), jnp.float32),
            pltpu.VMEM((2, N, NHID2), jnp.float32),
            pltpu.SemaphoreType.DMA((3,)),
            pltpu.SemaphoreType.DMA((4,)),
        ],
    )

    out = pl.pallas_call(
        _gcn_kernel,
        grid_spec=grid_spec,
        out_shape=jax.ShapeDtypeStruct((B, 1), jnp.float32),
    )(length, x, adj, W1, b1r, W2, b2r, Wlin, blinr)
    return out


# speedup vs baseline: 1.0304x; 1.0304x over previous
import jax
import jax.numpy as jnp
from jax.experimental import pallas as pl
from jax.experimental.pallas import tpu as pltpu

B, N, NFEAT, NHID1, NHID2 = 8, 2048, 128, 64, 32
ROW_TILE = 512
CHUNK = 512


def _gcn_kernel(length_ref, x_ref, adj_hbm, W1_ref, b1_ref, W2_ref, b2_ref,
                Wlin_ref, blin_ref, out_ref, adjv, s2_scr, sems, qsems):
    i = pl.program_id(0)

    @pl.when(i == 0)
    def _():
        for q in range(4):
            pltpu.make_async_copy(adj_hbm.at[0, pl.ds(q * CHUNK, CHUNK)],
                                  adjv.at[0, pl.ds(q * CHUNK, CHUNK)],
                                  qsems.at[q]).start()

    @pl.when(i + 1 < B)
    def _():
        pltpu.make_async_copy(adj_hbm.at[i + 1], adjv.at[(i + 1) % 3],
                              sems.at[(i + 1) % 3]).start()

    @pl.when(i > 0)
    def _():
        g = i - 1
        L = length_ref[g]
        s2 = s2_scr[g % 2]
        n_tiles = (L + ROW_TILE - 1) // ROW_TILE
        tile_iota = jax.lax.broadcasted_iota(jnp.int32, (ROW_TILE, 1), 0)

        def tile_body(t, acc):
            r0 = t * ROW_TILE
            z = jnp.dot(adjv[g % 3, pl.ds(r0, ROW_TILE), :], s2,
                        preferred_element_type=jnp.float32) + b2_ref[:]
            z = jnp.maximum(z, 0.0)
            z = jnp.where(tile_iota + r0 < L, z, 0.0)
            return acc + jnp.sum(z, axis=0, keepdims=True)

        pooled = jax.lax.fori_loop(
            0, n_tiles, tile_body, jnp.zeros((1, NHID2), jnp.float32))
        pooled = pooled / L.astype(jnp.float32)
        out_ref[pl.ds(g, 1), :] = jnp.dot(
            pooled, Wlin_ref[:],
            preferred_element_type=jnp.float32) + blin_ref[:]

    @pl.when(i == 0)
    def _():
        s1 = jnp.dot(x_ref[0], W1_ref[:], preferred_element_type=jnp.float32)
        for q in range(4):
            pltpu.make_async_copy(adj_hbm.at[0, pl.ds(q * CHUNK, CHUNK)],
                                  adjv.at[0, pl.ds(q * CHUNK, CHUNK)],
                                  qsems.at[q]).wait()
            hq = jnp.dot(adjv[0, q * CHUNK:(q + 1) * CHUNK, :], s1,
                         preferred_element_type=jnp.float32) + b1_ref[:]
            hq = jnp.maximum(hq, 0.0)
            s2_scr[0, q * CHUNK:(q + 1) * CHUNK, :] = jnp.dot(
                hq, W2_ref[:], preferred_element_type=jnp.float32)

    @pl.when(jnp.logical_and(i >= 1, i < B))
    def _():
        s1 = jnp.dot(x_ref[0], W1_ref[:], preferred_element_type=jnp.float32)
        pltpu.make_async_copy(adj_hbm.at[i], adjv.at[i % 3],
                              sems.at[i % 3]).wait()
        h = jnp.dot(adjv[i % 3], s1,
                    preferred_element_type=jnp.float32) + b1_ref[:]
        h = jnp.maximum(h, 0.0)
        s2_scr[i % 2] = jnp.dot(h, W2_ref[:],
                                preferred_element_type=jnp.float32)


def kernel(x, adj, length, W1, b1, W2, b2, Wlin, blin):
    b1r = b1.reshape(1, NHID1)
    b2r = b2.reshape(1, NHID2)
    blinr = blin.reshape(1, 1)

    grid_spec = pltpu.PrefetchScalarGridSpec(
        num_scalar_prefetch=1,
        grid=(B + 1,),
        in_specs=[
            pl.BlockSpec((1, N, NFEAT), lambda i, L: (jnp.minimum(i, B - 1), 0, 0)),
            pl.BlockSpec(memory_space=pl.ANY),
            pl.BlockSpec((NFEAT, NHID1), lambda i, L: (0, 0)),
            pl.BlockSpec((1, NHID1), lambda i, L: (0, 0)),
            pl.BlockSpec((NHID1, NHID2), lambda i, L: (0, 0)),
            pl.BlockSpec((1, NHID2), lambda i, L: (0, 0)),
            pl.BlockSpec((NHID2, 1), lambda i, L: (0, 0)),
            pl.BlockSpec((1, 1), lambda i, L: (0, 0)),
        ],
        out_specs=pl.BlockSpec((B, 1), lambda i, L: (0, 0)),
        scratch_shapes=[
            pltpu.VMEM((3, N, N), jnp.float32),
            pltpu.VMEM((2, N, NHID2), jnp.float32),
            pltpu.SemaphoreType.DMA((3,)),
            pltpu.SemaphoreType.DMA((4,)),
        ],
    )

    out = pl.pallas_call(
        _gcn_kernel,
        grid_spec=grid_spec,
        out_shape=jax.ShapeDtypeStruct((B, 1), jnp.float32),
    )(length, x, adj, W1, b1r, W2, b2r, Wlin, blinr)
    return out


# speedup vs baseline: 1.0382x; 1.0075x over previous
import jax
import jax.numpy as jnp
from jax.experimental import pallas as pl
from jax.experimental.pallas import tpu as pltpu

B, N, NFEAT, NHID1, NHID2 = 8, 2048, 128, 64, 32
ROW_TILE = 1024
CHUNK = 512


def _gcn_kernel(length_ref, x_ref, adj_hbm, W1_ref, b1_ref, W2_ref, b2_ref,
                Wlin_ref, blin_ref, out_ref, adjv, s2_scr, sems, qsems):
    i = pl.program_id(0)

    @pl.when(i == 0)
    def _():
        for q in range(4):
            pltpu.make_async_copy(adj_hbm.at[0, pl.ds(q * CHUNK, CHUNK)],
                                  adjv.at[0, pl.ds(q * CHUNK, CHUNK)],
                                  qsems.at[q]).start()

    @pl.when(i + 1 < B)
    def _():
        pltpu.make_async_copy(adj_hbm.at[i + 1], adjv.at[(i + 1) % 3],
                              sems.at[(i + 1) % 3]).start()

    @pl.when(i > 0)
    def _():
        g = i - 1
        L = length_ref[g]
        s2 = s2_scr[g % 2]
        n_tiles = (L + ROW_TILE - 1) // ROW_TILE
        tile_iota = jax.lax.broadcasted_iota(jnp.int32, (ROW_TILE, 1), 0)

        def tile_body(t, acc):
            r0 = t * ROW_TILE
            z = jnp.dot(adjv[g % 3, pl.ds(r0, ROW_TILE), :], s2,
                        preferred_element_type=jnp.float32) + b2_ref[:]
            z = jnp.maximum(z, 0.0)
            z = jnp.where(tile_iota + r0 < L, z, 0.0)
            return acc + jnp.sum(z, axis=0, keepdims=True)

        pooled = jax.lax.fori_loop(
            0, n_tiles, tile_body, jnp.zeros((1, NHID2), jnp.float32))
        pooled = pooled / L.astype(jnp.float32)
        out_ref[pl.ds(g, 1), :] = jnp.dot(
            pooled, Wlin_ref[:],
            preferred_element_type=jnp.float32) + blin_ref[:]

    @pl.when(i == 0)
    def _():
        s1 = jnp.dot(x_ref[0], W1_ref[:], preferred_element_type=jnp.float32)
        for q in range(4):
            pltpu.make_async_copy(adj_hbm.at[0, pl.ds(q * CHUNK, CHUNK)],
                                  adjv.at[0, pl.ds(q * CHUNK, CHUNK)],
                                  qsems.at[q]).wait()
            hq = jnp.dot(adjv[0, q * CHUNK:(q + 1) * CHUNK, :], s1,
                         preferred_element_type=jnp.float32) + b1_ref[:]
            hq = jnp.maximum(hq, 0.0)
            s2_scr[0, q * CHUNK:(q + 1) * CHUNK, :] = jnp.dot(
                hq, W2_ref[:], preferred_element_type=jnp.float32)

    @pl.when(jnp.logical_and(i >= 1, i < B))
    def _():
        s1 = jnp.dot(x_ref[0], W1_ref[:], preferred_element_type=jnp.float32)
        pltpu.make_async_copy(adj_hbm.at[i], adjv.at[i % 3],
                              sems.at[i % 3]).wait()
        h = jnp.dot(adjv[i % 3], s1,
                    preferred_element_type=jnp.float32) + b1_ref[:]
        h = jnp.maximum(h, 0.0)
        s2_scr[i % 2] = jnp.dot(h, W2_ref[:],
                                preferred_element_type=jnp.float32)


def kernel(x, adj, length, W1, b1, W2, b2, Wlin, blin):
    b1r = b1.reshape(1, NHID1)
    b2r = b2.reshape(1, NHID2)
    blinr = blin.reshape(1, 1)

    grid_spec = pltpu.PrefetchScalarGridSpec(
        num_scalar_prefetch=1,
        grid=(B + 1,),
        in_specs=[
            pl.BlockSpec((1, N, NFEAT), lambda i, L: (jnp.minimum(i, B - 1), 0, 0)),
            pl.BlockSpec(memory_space=pl.ANY),
            pl.BlockSpec((NFEAT, NHID1), lambda i, L: (0, 0)),
            pl.BlockSpec((1, NHID1), lambda i, L: (0, 0)),
            pl.BlockSpec((NHID1, NHID2), lambda i, L: (0, 0)),
            pl.BlockSpec((1, NHID2), lambda i, L: (0, 0)),
            pl.BlockSpec((NHID2, 1), lambda i, L: (0, 0)),
            pl.BlockSpec((1, 1), lambda i, L: (0, 0)),
        ],
        out_specs=pl.BlockSpec((B, 1), lambda i, L: (0, 0)),
        scratch_shapes=[
            pltpu.VMEM((3, N, N), jnp.float32),
            pltpu.VMEM((2, N, NHID2), jnp.float32),
            pltpu.SemaphoreType.DMA((3,)),
            pltpu.SemaphoreType.DMA((4,)),
        ],
    )

    out = pl.pallas_call(
        _gcn_kernel,
        grid_spec=grid_spec,
        out_shape=jax.ShapeDtypeStruct((B, 1), jnp.float32),
    )(length, x, adj, W1, b1r, W2, b2r, Wlin, blinr)
    return out
